# baseline (speedup 1.0000x reference)
import functools

import jax
import jax.numpy as jnp
from jax import lax
from jax.experimental import pallas as pl

EMB = 128
NUM_GRAPHS = 256
CUTOFF = 10.0
NUM_GAUSS = 50
NUM_BLOCKS = 2
K = 32
RCHUNK = 200


def _topk_body(d_ref, nbr_ref, ew_ref, val_ref, *, rchunk, n):
    score = d_ref[...]
    col = lax.broadcasted_iota(jnp.int32, (rchunk, n), 1)
    nbrs, ews, vals = [], [], []
    for _ in range(K):
        m = jnp.min(score, axis=1, keepdims=True)
        hit = score == m
        idx = jnp.min(jnp.where(hit, col, n), axis=1, keepdims=True)
        ok = m < 1e30
        nbrs.append(idx)
        ews.append(jnp.where(ok, m, 0.0))
        vals.append(ok.astype(jnp.float32))
        score = jnp.where(col == idx, jnp.inf, score)
    nbr_ref[...] = jnp.concatenate(nbrs, axis=1)
    ew_ref[...] = jnp.concatenate(ews, axis=1)
    val_ref[...] = jnp.concatenate(vals, axis=1)


def _radius_pallas(positions, batch, rchunk=RCHUNK):
    n = positions.shape[0]
    chunk = 1000 if n % 1000 == 0 else n
    nchunks = n // chunk
    sq = (positions ** 2).sum(1)
    idxs = jnp.arange(n, dtype=jnp.int32)

    def f(args):
        pc, bc, ic = args
        d2 = sq[ic][:, None] + sq[None, :] - 2.0 * pc @ positions.T
        d = jnp.sqrt(jnp.maximum(d2, 0.0))
        mask = (bc[:, None] == batch[None, :]) & (d < CUTOFF)
        mask = mask.at[jnp.arange(chunk), ic].set(False)
        return jnp.where(mask, d, jnp.inf)

    d_inf = jax.lax.map(f, (positions.reshape(nchunks, chunk, 3),
                            batch.reshape(nchunks, chunk),
                            idxs.reshape(nchunks, chunk))).reshape(n, n)

    grid = n // rchunk
    body = functools.partial(_topk_body, rchunk=rchunk, n=n)
    return pl.pallas_call(
        body,
        grid=(grid,),
        in_specs=[pl.BlockSpec((rchunk, n), lambda i: (i, 0))],
        out_specs=[
            pl.BlockSpec((rchunk, K), lambda i: (i, 0)),
            pl.BlockSpec((rchunk, K), lambda i: (i, 0)),
            pl.BlockSpec((rchunk, K), lambda i: (i, 0)),
        ],
        out_shape=[
            jax.ShapeDtypeStruct((n, K), jnp.int32),
            jax.ShapeDtypeStruct((n, K), jnp.float32),
            jax.ShapeDtypeStruct((n, K), jnp.float32),
        ],
    )(d_inf)


def _ssp(v):
    return jax.nn.softplus(v) - jnp.log(2.0)


def _bn(v, g, b, eps=1e-5):
    m = v.mean(0)
    var = v.var(0)
    return (v - m) / jnp.sqrt(var + eps) * g + b


def kernel(x, edge_index, edge_attr, positions, batch, params):
    N = x.shape[0]
    nbr, ew_k, val_k = _radius_pallas(positions, batch)
    src3 = nbr.reshape(-1)
    tgt3 = jnp.repeat(jnp.arange(N, dtype=jnp.int32), K)
    mask3 = val_k.reshape(-1) > 0.5
    row3, col3 = src3, tgt3
    virt_idx = (jnp.searchsorted(batch, jnp.arange(NUM_GRAPHS, dtype=batch.dtype),
                                 side='right') - 1).astype(jnp.int32)
    ar = jnp.arange(N, dtype=jnp.int32)
    src = jnp.concatenate([edge_index[0].astype(jnp.int32), ar])
    dst = jnp.concatenate([edge_index[1].astype(jnp.int32), ar])
    ea0 = jnp.concatenate([edge_attr[:, 0], jnp.full((N,), 4, edge_attr.dtype)])
    ea1 = jnp.concatenate([edge_attr[:, 1], jnp.zeros((N,), edge_attr.dtype)])

    x2 = params['emb2d'][x]
    x3 = params['emb3d'][x]
    prev2, prev3 = x2, x3
    ew = jnp.sqrt(((positions[row3] - positions[col3]) ** 2).sum(-1))
    ew = jnp.where(mask3, ew, 0.0)
    offsets = jnp.linspace(0.0, CUTOFF, NUM_GAUSS)
    coeff = -0.5 / (offsets[1] - offsets[0]) ** 2
    ea3 = jnp.exp(coeff * (ew[:, None] - offsets[None, :]) ** 2)
    C = 0.5 * (jnp.cos(ew * jnp.pi / CUTOFF) + 1.0)
    C = jnp.where(mask3, C, 0.0)
    sch = params['sch']
    W = (_ssp(ea3 @ sch['mW1'] + sch['mb1']) @ sch['mW2'] + sch['mb2']) * C[:, None]
    for i in range(NUM_BLOCKS):
        g = params['gin'][i]
        e_emb = g['e1'][ea0] + g['e2'][ea1]
        msg = x2[src] + e_emb
        agg = jnp.zeros((N, EMB), jnp.float32).at[dst].add(msg)
        h2 = jax.nn.relu(agg @ g['W1'] + g['b1']) @ g['W2'] + g['b2']
        h2 = jax.nn.relu(_bn(h2, params['ng'], params['nb']))
        x2 = h2 + prev2
        xx = x3 @ sch['lin1W']
        agg3 = jnp.zeros((N, EMB), jnp.float32).at[col3].add(xx[row3] * W)
        xx = agg3 @ sch['lin2W'] + sch['lin2b']
        h3 = _ssp(xx) @ sch['linW'] + sch['linb']
        h3 = jax.nn.relu(_bn(h3, params['ng'], params['nb']))
        x3 = h3 + prev3
        v2 = x2[virt_idx]
        v3 = x3[virt_idx]
        it = jnp.concatenate([v2, v3], axis=-1)
        it = it @ params['iW1'] + params['ib1']
        it = jax.nn.relu(_bn(it, params['ibg'], params['ibb']))
        it = it @ params['iW2'] + params['ib2']
        x2 = x2.at[virt_idx].set(it[:, :EMB])
        x3 = x3.at[virt_idx].set(it[:, EMB:])
        prev2, prev3 = x2, x3
    return it


# speedup vs baseline: 1.0013x; 1.0013x over previous
import functools

import jax
import jax.numpy as jnp
from jax import lax
from jax.experimental import pallas as pl

EMB = 128
NUM_GRAPHS = 256
CUTOFF = 10.0
NUM_GAUSS = 50
NUM_BLOCKS = 2
K = 32
RCHUNK = 200


def _topk_body(d_ref, nbr_ref, val_ref, *, rchunk, n):
    score = d_ref[...]
    col = lax.broadcasted_iota(jnp.int32, (rchunk, n), 1)
    nvalid = jnp.sum((score < jnp.inf).astype(jnp.int32), axis=1, keepdims=True)
    nbrs = []
    for _ in range(K):
        m = jnp.min(score, axis=1, keepdims=True)
        idx = jnp.min(jnp.where(score == m, col, n), axis=1, keepdims=True)
        nbrs.append(idx)
        score = jnp.where(col == idx, jnp.inf, score)
    nbr_ref[...] = jnp.concatenate(nbrs, axis=1)
    jcol = lax.broadcasted_iota(jnp.int32, (rchunk, K), 1)
    val_ref[...] = (jcol < nvalid).astype(jnp.float32)


def _radius_pallas(positions, batch, rchunk=RCHUNK):
    n = positions.shape[0]
    chunk = 1000 if n % 1000 == 0 else n
    nchunks = n // chunk
    sq = (positions ** 2).sum(1)
    idxs = jnp.arange(n, dtype=jnp.int32)

    def f(args):
        pc, bc, ic = args
        d2 = sq[ic][:, None] + sq[None, :] - 2.0 * pc @ positions.T
        d = jnp.sqrt(jnp.maximum(d2, 0.0))
        mask = (bc[:, None] == batch[None, :]) & (d < CUTOFF)
        mask = mask.at[jnp.arange(chunk), ic].set(False)
        return jnp.where(mask, d, jnp.inf)

    d_inf = jax.lax.map(f, (positions.reshape(nchunks, chunk, 3),
                            batch.reshape(nchunks, chunk),
                            idxs.reshape(nchunks, chunk))).reshape(n, n)

    grid = n // rchunk
    body = functools.partial(_topk_body, rchunk=rchunk, n=n)
    return pl.pallas_call(
        body,
        grid=(grid,),
        in_specs=[pl.BlockSpec((rchunk, n), lambda i: (i, 0))],
        out_specs=[
            pl.BlockSpec((rchunk, K), lambda i: (i, 0)),
            pl.BlockSpec((rchunk, K), lambda i: (i, 0)),
        ],
        out_shape=[
            jax.ShapeDtypeStruct((n, K), jnp.int32),
            jax.ShapeDtypeStruct((n, K), jnp.float32),
        ],
    )(d_inf)


def _ssp(v):
    return jax.nn.softplus(v) - jnp.log(2.0)


def _bn(v, g, b, eps=1e-5):
    m = v.mean(0)
    var = v.var(0)
    return (v - m) / jnp.sqrt(var + eps) * g + b


def kernel(x, edge_index, edge_attr, positions, batch, params):
    N = x.shape[0]
    nbr, val_k = _radius_pallas(positions, batch)
    src3 = nbr.reshape(-1)
    tgt3 = jnp.repeat(jnp.arange(N, dtype=jnp.int32), K)
    mask3 = val_k.reshape(-1) > 0.5
    row3, col3 = src3, tgt3
    virt_idx = (jnp.searchsorted(batch, jnp.arange(NUM_GRAPHS, dtype=batch.dtype),
                                 side='right') - 1).astype(jnp.int32)
    ar = jnp.arange(N, dtype=jnp.int32)
    src = jnp.concatenate([edge_index[0].astype(jnp.int32), ar])
    dst = jnp.concatenate([edge_index[1].astype(jnp.int32), ar])
    ea0 = jnp.concatenate([edge_attr[:, 0], jnp.full((N,), 4, edge_attr.dtype)])
    ea1 = jnp.concatenate([edge_attr[:, 1], jnp.zeros((N,), edge_attr.dtype)])

    x2 = params['emb2d'][x]
    x3 = params['emb3d'][x]
    prev2, prev3 = x2, x3
    ew = jnp.sqrt(((positions[row3] - positions[col3]) ** 2).sum(-1))
    ew = jnp.where(mask3, ew, 0.0)
    offsets = jnp.linspace(0.0, CUTOFF, NUM_GAUSS)
    coeff = -0.5 / (offsets[1] - offsets[0]) ** 2
    ea3 = jnp.exp(coeff * (ew[:, None] - offsets[None, :]) ** 2)
    C = 0.5 * (jnp.cos(ew * jnp.pi / CUTOFF) + 1.0)
    C = jnp.where(mask3, C, 0.0)
    sch = params['sch']
    W = (_ssp(ea3 @ sch['mW1'] + sch['mb1']) @ sch['mW2'] + sch['mb2']) * C[:, None]
    for i in range(NUM_BLOCKS):
        g = params['gin'][i]
        e_emb = g['e1'][ea0] + g['e2'][ea1]
        msg = x2[src] + e_emb
        agg = jnp.zeros((N, EMB), jnp.float32).at[dst].add(msg)
        h2 = jax.nn.relu(agg @ g['W1'] + g['b1']) @ g['W2'] + g['b2']
        h2 = jax.nn.relu(_bn(h2, params['ng'], params['nb']))
        x2 = h2 + prev2
        xx = x3 @ sch['lin1W']
        agg3 = jnp.zeros((N, EMB), jnp.float32).at[col3].add(xx[row3] * W)
        xx = agg3 @ sch['lin2W'] + sch['lin2b']
        h3 = _ssp(xx) @ sch['linW'] + sch['linb']
        h3 = jax.nn.relu(_bn(h3, params['ng'], params['nb']))
        x3 = h3 + prev3
        v2 = x2[virt_idx]
        v3 = x3[virt_idx]
        it = jnp.concatenate([v2, v3], axis=-1)
        it = it @ params['iW1'] + params['ib1']
        it = jax.nn.relu(_bn(it, params['ibg'], params['ibb']))
        it = it @ params['iW2'] + params['ib2']
        x2 = x2.at[virt_idx].set(it[:, :EMB])
        x3 = x3.at[virt_idx].set(it[:, EMB:])
        prev2, prev3 = x2, x3
    return it


# speedup vs baseline: 1.2889x; 1.2872x over previous
import functools

import jax
import jax.numpy as jnp
from jax import lax
from jax.experimental import pallas as pl
from jax.experimental.pallas import tpu as pltpu
from jax.experimental.pallas import tpu_sc as plsc

EMB = 128
NUM_GRAPHS = 256
CUTOFF = 10.0
NUM_GAUSS = 50
NUM_BLOCKS = 2
K = 32
RCHUNK = 200


def _topk_body(d_ref, nbr_ref, val_ref, *, rchunk, n):
    score = d_ref[...]
    col = lax.broadcasted_iota(jnp.int32, (rchunk, n), 1)
    nvalid = jnp.sum((score < jnp.inf).astype(jnp.int32), axis=1, keepdims=True)
    nbrs = []
    for _ in range(K):
        m = jnp.min(score, axis=1, keepdims=True)
        idx = jnp.min(jnp.where(score == m, col, n), axis=1, keepdims=True)
        nbrs.append(idx)
        score = jnp.where(col == idx, jnp.inf, score)
    nbr_ref[...] = jnp.concatenate(nbrs, axis=1)
    jcol = lax.broadcasted_iota(jnp.int32, (rchunk, K), 1)
    val_ref[...] = (jcol < nvalid).astype(jnp.float32)


def _radius_pallas(positions, batch, rchunk=RCHUNK):
    n = positions.shape[0]
    chunk = 1000 if n % 1000 == 0 else n
    nchunks = n // chunk
    sq = (positions ** 2).sum(1)
    idxs = jnp.arange(n, dtype=jnp.int32)

    def f(args):
        pc, bc, ic = args
        d2 = sq[ic][:, None] + sq[None, :] - 2.0 * pc @ positions.T
        d = jnp.sqrt(jnp.maximum(d2, 0.0))
        mask = (bc[:, None] == batch[None, :]) & (d < CUTOFF)
        mask = mask.at[jnp.arange(chunk), ic].set(False)
        return jnp.where(mask, d, jnp.inf)

    d_inf = jax.lax.map(f, (positions.reshape(nchunks, chunk, 3),
                            batch.reshape(nchunks, chunk),
                            idxs.reshape(nchunks, chunk))).reshape(n, n)

    grid = n // rchunk
    body = functools.partial(_topk_body, rchunk=rchunk, n=n)
    return pl.pallas_call(
        body,
        grid=(grid,),
        in_specs=[pl.BlockSpec((rchunk, n), lambda i: (i, 0))],
        out_specs=[
            pl.BlockSpec((rchunk, K), lambda i: (i, 0)),
            pl.BlockSpec((rchunk, K), lambda i: (i, 0)),
        ],
        out_shape=[
            jax.ShapeDtypeStruct((n, K), jnp.int32),
            jax.ShapeDtypeStruct((n, K), jnp.float32),
        ],
    )(d_inf)


def _ssp(v):
    return jax.nn.softplus(v) - jnp.log(2.0)


def _bn(v, g, b, eps=1e-5):
    m = v.mean(0)
    var = v.var(0)
    return (v - m) / jnp.sqrt(var + eps) * g + b


_NPW = 320
_SCCH = 4


def _schnet_agg_sc(xx, W, nbr_flat):
    n = xx.shape[0]
    mesh = plsc.VectorSubcoreMesh(core_axis_name="c", subcore_axis_name="s")

    @functools.partial(
        pl.kernel, mesh=mesh,
        out_type=jax.ShapeDtypeStruct((n, EMB), jnp.float32),
        scratch_types=[
            pltpu.VMEM((_SCCH * K,), jnp.int32),
            pltpu.VMEM((_SCCH * K, EMB), jnp.float32),
            pltpu.VMEM((_SCCH * K, EMB), jnp.float32),
            pltpu.VMEM((_SCCH, EMB), jnp.float32),
            pltpu.SemaphoreType.DMA,
        ],
    )
    def k(xx_hbm, w_hbm, nbr_hbm, out_hbm, idx_v, rows_v, w_v, acc_v, sem):
        wid = lax.axis_index("s") * 2 + lax.axis_index("c")
        base = wid * _NPW
        nchunks = jnp.maximum(0, (jnp.minimum(n, base + _NPW) - base)) // _SCCH

        def chunk_body(ci, carry):
            nb = base + ci * _SCCH
            eb = nb * K
            pltpu.sync_copy(nbr_hbm.at[pl.ds(eb, _SCCH * K)], idx_v)
            pltpu.async_copy(xx_hbm.at[idx_v], rows_v, sem).wait()
            pltpu.sync_copy(w_hbm.at[pl.ds(eb, _SCCH * K)], w_v)
            for nn in range(_SCCH):
                for f in range(EMB // 16):
                    a = jnp.zeros((16,), jnp.float32)
                    for j in range(K):
                        r = rows_v[nn * K + j, pl.ds(f * 16, 16)]
                        w = w_v[nn * K + j, pl.ds(f * 16, 16)]
                        a = a + r * w
                    acc_v[nn, pl.ds(f * 16, 16)] = a
            pltpu.sync_copy(acc_v, out_hbm.at[pl.ds(nb, _SCCH)])
            return carry

        lax.fori_loop(0, nchunks, chunk_body, 0)

    return k(xx, W, nbr_flat)


def kernel(x, edge_index, edge_attr, positions, batch, params):
    N = x.shape[0]
    nbr, val_k = _radius_pallas(positions, batch)
    src3 = nbr.reshape(-1)
    tgt3 = jnp.repeat(jnp.arange(N, dtype=jnp.int32), K)
    mask3 = val_k.reshape(-1) > 0.5
    row3, col3 = src3, tgt3
    virt_idx = (jnp.searchsorted(batch, jnp.arange(NUM_GRAPHS, dtype=batch.dtype),
                                 side='right') - 1).astype(jnp.int32)
    ar = jnp.arange(N, dtype=jnp.int32)
    src = jnp.concatenate([edge_index[0].astype(jnp.int32), ar])
    dst = jnp.concatenate([edge_index[1].astype(jnp.int32), ar])
    ea0 = jnp.concatenate([edge_attr[:, 0], jnp.full((N,), 4, edge_attr.dtype)])
    ea1 = jnp.concatenate([edge_attr[:, 1], jnp.zeros((N,), edge_attr.dtype)])

    x2 = params['emb2d'][x]
    x3 = params['emb3d'][x]
    prev2, prev3 = x2, x3
    ew = jnp.sqrt(((positions[row3] - positions[col3]) ** 2).sum(-1))
    ew = jnp.where(mask3, ew, 0.0)
    offsets = jnp.linspace(0.0, CUTOFF, NUM_GAUSS)
    coeff = -0.5 / (offsets[1] - offsets[0]) ** 2
    ea3 = jnp.exp(coeff * (ew[:, None] - offsets[None, :]) ** 2)
    C = 0.5 * (jnp.cos(ew * jnp.pi / CUTOFF) + 1.0)
    C = jnp.where(mask3, C, 0.0)
    sch = params['sch']
    W = (_ssp(ea3 @ sch['mW1'] + sch['mb1']) @ sch['mW2'] + sch['mb2']) * C[:, None]
    cnt0 = jnp.zeros((N, 6), jnp.float32).at[dst, ea0].add(1.0)
    cnt1 = jnp.zeros((N, 3), jnp.float32).at[dst, ea1].add(1.0)
    for i in range(NUM_BLOCKS):
        g = params['gin'][i]
        agg = jnp.zeros((N, EMB), jnp.float32).at[dst].add(x2[src])
        agg = agg + jnp.dot(cnt0, g['e1'], precision=lax.Precision.HIGHEST) \
                  + jnp.dot(cnt1, g['e2'], precision=lax.Precision.HIGHEST)
        h2 = jax.nn.relu(agg @ g['W1'] + g['b1']) @ g['W2'] + g['b2']
        h2 = jax.nn.relu(_bn(h2, params['ng'], params['nb']))
        x2 = h2 + prev2
        xx = x3 @ sch['lin1W']
        agg3 = _schnet_agg_sc(xx, W, src3)
        xx = agg3 @ sch['lin2W'] + sch['lin2b']
        h3 = _ssp(xx) @ sch['linW'] + sch['linb']
        h3 = jax.nn.relu(_bn(h3, params['ng'], params['nb']))
        x3 = h3 + prev3
        v2 = x2[virt_idx]
        v3 = x3[virt_idx]
        it = jnp.concatenate([v2, v3], axis=-1)
        it = it @ params['iW1'] + params['ib1']
        it = jax.nn.relu(_bn(it, params['ibg'], params['ibb']))
        it = it @ params['iW2'] + params['ib2']
        x2 = x2.at[virt_idx].set(it[:, :EMB])
        x3 = x3.at[virt_idx].set(it[:, EMB:])
        prev2, prev3 = x2, x3
    return it
